# Initial kernel scaffold; baseline (speedup 1.0000x reference)
import jax
import jax.numpy as jnp
from jax.experimental import pallas as pl


def kernel(x, edge_index, edge_type, W1_rel, W1_root, b1, W2_rel, W2_root, b2, W3_rel, W3_root, b3, W4_rel, W4_root, b4):
    raise NotImplementedError("write your pallas kernel here")


# speedup vs baseline: 10.0973x; 10.0973x over previous
import functools

import jax
import jax.numpy as jnp
from jax import lax
from jax.experimental import pallas as pl
from jax.experimental.pallas import tpu as pltpu
from jax.experimental.pallas import tpu_sc as plsc

_N = 10000
_E = 320000
_R = 8
_LANES = 16

_NC = 2
_NS = 16
_NW = _NC * _NS
_EPW = _E // _NW
_K = 80
_NCHUNK = _EPW // _K

_NP = 10240
_RT = _NP // _NS
_ZP = 5
_ZR = _RT // _ZP

_SP = 81920
_RT2 = _SP // _NS
_ZR2 = _RT2 // _ZP

_mesh = lambda: plsc.VectorSubcoreMesh(core_axis_name="c", subcore_axis_name="s")
_sc_params = lambda: pltpu.CompilerParams(use_tc_tiling_on_sc=False)


def _sc_counts(seg):
  @functools.partial(
      pl.kernel,
      mesh=_mesh(),
      compiler_params=_sc_params(),
      out_type=jax.ShapeDtypeStruct((_NC, _SP, _LANES), jnp.float32),
      scratch_types=[
          pltpu.VMEM_SHARED((_SP, _LANES), jnp.float32),
          pltpu.VMEM((_ZR2, _LANES), jnp.float32),
          pltpu.VMEM((_K, _LANES), jnp.float32),
          pltpu.VMEM((_K,), jnp.int32),
      ],
  )
  def k(seg_hbm, out_hbm, acc, zb, ones, si):
    c = lax.axis_index("c")
    s = lax.axis_index("s")
    wid = c * _NS + s
    zv = jnp.zeros((_LANES,), jnp.float32)
    ov = jnp.ones((_LANES,), jnp.float32)

    def fill(i, carry):
      zb[i, :] = zv
      return carry

    lax.fori_loop(0, _ZR2, fill, 0)

    def fillo(i, carry):
      ones[i, :] = ov
      return carry

    lax.fori_loop(0, _K, fillo, 0)

    for p in range(_ZP):
      pltpu.sync_copy(zb, acc.at[pl.ds(s * _RT2 + p * _ZR2, _ZR2)])
    plsc.subcore_barrier()

    def chunk(ci, carry):
      base = wid * _EPW + ci * _K
      pltpu.sync_copy(seg_hbm.at[pl.ds(base, _K)], si)
      pltpu.sync_copy(ones, acc.at[si], add=True)
      return carry

    lax.fori_loop(0, _NCHUNK, chunk, 0)
    plsc.subcore_barrier()

    for p in range(_ZP):
      r0 = s * _RT2 + p * _ZR2
      pltpu.sync_copy(acc.at[pl.ds(r0, _ZR2)], zb)
      pltpu.sync_copy(zb, out_hbm.at[c, pl.ds(r0, _ZR2)])

  return k(seg)


def _sc_wedge(c0, c1, seg):
  @functools.partial(
      pl.kernel,
      mesh=_mesh(),
      compiler_params=_sc_params(),
      out_type=jax.ShapeDtypeStruct((_E, _LANES), jnp.float32),
      scratch_types=[
          pltpu.VMEM((_K, _LANES), jnp.float32),
          pltpu.VMEM((_K, _LANES), jnp.float32),
          pltpu.VMEM((_K,), jnp.int32),
          pltpu.SemaphoreType.DMA,
          pltpu.SemaphoreType.DMA,
      ],
  )
  def k(c0_hbm, c1_hbm, seg_hbm, out_hbm, g0, g1, si, sem0, sem1):
    c = lax.axis_index("c")
    s = lax.axis_index("s")
    wid = c * _NS + s

    def chunk(ci, carry):
      base = wid * _EPW + ci * _K
      pltpu.sync_copy(seg_hbm.at[pl.ds(base, _K)], si)
      cp0 = pltpu.async_copy(c0_hbm.at[si], g0, sem0)
      cp1 = pltpu.async_copy(c1_hbm.at[si], g1, sem1)
      cp0.wait()
      cp1.wait()

      def row(i, carry2):
        g0[i, :] = 1.0 / jnp.maximum(g0[i, :] + g1[i, :], 1.0)
        return carry2

      lax.fori_loop(0, _K, row, 0)
      pltpu.sync_copy(g0, out_hbm.at[pl.ds(base, _K)])
      return carry

    lax.fori_loop(0, _NCHUNK, chunk, 0)

  return k(c0, c1, seg)


def _sc_edge_pass(mm, wedge, gidx, dst, width):
  nsub = width // _LANES

  @functools.partial(
      pl.kernel,
      mesh=_mesh(),
      compiler_params=_sc_params(),
      out_type=jax.ShapeDtypeStruct((_NC, _NP, width), jnp.float32),
      scratch_types=[
          pltpu.VMEM_SHARED((_NP, width), jnp.float32),
          pltpu.VMEM((_ZR, width), jnp.float32),
          pltpu.VMEM((_K, width), jnp.float32),
          pltpu.VMEM((_K, _LANES), jnp.float32),
          pltpu.VMEM((_K,), jnp.int32),
          pltpu.VMEM((_K,), jnp.int32),
          pltpu.SemaphoreType.DMA,
      ],
  )
  def k(mm_hbm, wedge_hbm, gidx_hbm, dst_hbm, out_hbm, acc, zb, feat, wrow, gi, di, sem):
    c = lax.axis_index("c")
    s = lax.axis_index("s")
    wid = c * _NS + s
    zv = jnp.zeros((_LANES,), jnp.float32)

    def fill(i, carry):
      for j in range(nsub):
        zb[i, pl.ds(j * _LANES, _LANES)] = zv
      return carry

    lax.fori_loop(0, _ZR, fill, 0)
    for p in range(_ZP):
      pltpu.sync_copy(zb, acc.at[pl.ds(s * _RT + p * _ZR, _ZR)])
    plsc.subcore_barrier()

    def chunk(ci, carry):
      base = wid * _EPW + ci * _K
      pltpu.sync_copy(gidx_hbm.at[pl.ds(base, _K)], gi)
      pltpu.sync_copy(dst_hbm.at[pl.ds(base, _K)], di)
      pltpu.sync_copy(wedge_hbm.at[pl.ds(base, _K)], wrow)
      pltpu.async_copy(mm_hbm.at[gi], feat, sem).wait()

      def row(i, carry2):
        w = wrow[i, :]
        for j in range(nsub):
          feat[i, pl.ds(j * _LANES, _LANES)] = feat[i, pl.ds(j * _LANES, _LANES)] * w
        return carry2

      lax.fori_loop(0, _K, row, 0)
      pltpu.sync_copy(feat, acc.at[di], add=True)
      return carry

    lax.fori_loop(0, _NCHUNK, chunk, 0)
    plsc.subcore_barrier()

    for p in range(_ZP):
      r0 = s * _RT + p * _ZR
      pltpu.sync_copy(acc.at[pl.ds(r0, _ZR)], zb)
      pltpu.sync_copy(zb, out_hbm.at[c, pl.ds(r0, _ZR)])

  return k(mm, wedge, gidx, dst)


def _tc_transform(h, wstack):
  nine, din, dout = wstack.shape
  bn = 2000
  nb = _N // bn

  def body(h_ref, w_ref, o_ref):
    o_ref[...] = jnp.dot(h_ref[...], w_ref[0], preferred_element_type=jnp.float32)

  return pl.pallas_call(
      body,
      grid=(nine, nb),
      in_specs=[
          pl.BlockSpec((bn, din), lambda j, i: (i, 0)),
          pl.BlockSpec((1, din, dout), lambda j, i: (j, 0, 0)),
      ],
      out_specs=pl.BlockSpec((bn, dout), lambda j, i: (j * nb + i, 0)),
      out_shape=jax.ShapeDtypeStruct((nine * _N, dout), jnp.float32),
  )(h, wstack)


def _tc_fuse(a, mm, b2, relu):
  width = a.shape[-1]
  bn = 2000
  nb = _N // bn

  def body(a0_ref, a1_ref, r_ref, b_ref, o_ref):
    z = a0_ref[0] + a1_ref[0] + r_ref[...] + b_ref[0]
    if relu:
      o_ref[...] = jnp.maximum(z, 0.0)
    else:
      m = jnp.max(z, axis=-1, keepdims=True)
      e = jnp.exp(z - m)
      o_ref[...] = z - m - jnp.log(jnp.sum(e, axis=-1, keepdims=True))

  return pl.pallas_call(
      body,
      grid=(nb,),
      in_specs=[
          pl.BlockSpec((1, bn, width), lambda i: (0, i, 0)),
          pl.BlockSpec((1, bn, width), lambda i: (1, i, 0)),
          pl.BlockSpec((bn, width), lambda i: (8 * nb + i, 0)),
          pl.BlockSpec((1, width), lambda i: (0, 0)),
      ],
      out_specs=pl.BlockSpec((bn, width), lambda i: (i, 0)),
      out_shape=jax.ShapeDtypeStruct((_N, width), jnp.float32),
  )(a, a, mm, b2)


def kernel(x, edge_index, edge_type,
           W1_rel, W1_root, b1,
           W2_rel, W2_root, b2,
           W3_rel, W3_root, b3,
           W4_rel, W4_root, b4):
  src = edge_index[0].astype(jnp.int32)
  dst = edge_index[1].astype(jnp.int32)
  et = edge_type.astype(jnp.int32)
  gidx = et * _N + src
  seg = dst * _R + et

  cnt = _sc_counts(seg)
  wedge = _sc_wedge(cnt[0], cnt[1], seg)

  h = x.astype(jnp.float32)
  layers = [
      (W1_rel, W1_root, b1, True),
      (W2_rel, W2_root, b2, True),
      (W3_rel, W3_root, b3, True),
      (W4_rel, W4_root, b4, False),
  ]
  for w_rel, w_root, b, relu in layers:
    wstack = jnp.concatenate([w_rel, w_root[None]], axis=0)
    mm = _tc_transform(h, wstack)
    a = _sc_edge_pass(mm, wedge, gidx, dst, w_rel.shape[-1])
    h = _tc_fuse(a, mm, b.reshape(1, -1), relu)
  return h


# speedup vs baseline: 23.7201x; 2.3492x over previous
import functools

import jax
import jax.numpy as jnp
from jax import lax
from jax.experimental import pallas as pl
from jax.experimental.pallas import tpu as pltpu
from jax.experimental.pallas import tpu_sc as plsc

_N = 10000
_E = 320000
_R = 8
_LANES = 16

_NC = 2
_NS = 16
_NW = _NC * _NS
_EPW = _E // _NW
_K = 80
_NCHUNK = _EPW // _K

_NP = 10240
_RT = _NP // _NS
_ZP = 5
_ZR = _RT // _ZP

_SP = 81920
_RT2 = _SP // _NS
_ZR2 = _RT2 // _ZP

_mesh = lambda: plsc.VectorSubcoreMesh(core_axis_name="c", subcore_axis_name="s")
_sc_params = lambda: pltpu.CompilerParams(use_tc_tiling_on_sc=False)


def _sc_counts(seg2d):
  @functools.partial(
      pl.kernel,
      mesh=_mesh(),
      compiler_params=_sc_params(),
      out_type=jax.ShapeDtypeStruct((_NC, _SP, _LANES), jnp.float32),
      scratch_types=[
          pltpu.VMEM_SHARED((_SP, _LANES), jnp.float32),
          pltpu.VMEM((_ZR2, _LANES), jnp.float32),
          pltpu.VMEM((_K, _LANES), jnp.float32),
          pltpu.VMEM((_NCHUNK, _K), jnp.int32),
          pltpu.SemaphoreType.DMA,
      ],
  )
  def k(seg_hbm, out_hbm, acc, zb, ones, si, sem):
    c = lax.axis_index("c")
    s = lax.axis_index("s")
    wid = c * _NS + s
    zv = jnp.zeros((_LANES,), jnp.float32)
    ov = jnp.ones((_LANES,), jnp.float32)

    def fill(i, carry):
      zb[i, :] = zv
      return carry

    lax.fori_loop(0, _ZR2, fill, 0)

    def fillo(i, carry):
      ones[i, :] = ov
      return carry

    lax.fori_loop(0, _K, fillo, 0)

    pltpu.sync_copy(seg_hbm.at[pl.ds(wid * _NCHUNK, _NCHUNK)], si)
    for p in range(_ZP):
      pltpu.sync_copy(zb, acc.at[pl.ds(s * _RT2 + p * _ZR2, _ZR2)])
    plsc.subcore_barrier()

    def group(g, carry):
      for i in range(5):
        pltpu.async_copy(ones, acc.at[si.at[g * 5 + i]], sem, add=True)
      for i in range(5):
        pltpu.make_async_copy(ones, acc.at[si.at[0]], sem).wait()
      return carry

    lax.fori_loop(0, _NCHUNK // 5, group, 0)
    plsc.subcore_barrier()

    for p in range(_ZP):
      r0 = s * _RT2 + p * _ZR2
      pltpu.sync_copy(acc.at[pl.ds(r0, _ZR2)], zb)
      pltpu.sync_copy(zb, out_hbm.at[c, pl.ds(r0, _ZR2)])

  return k(seg2d)


def _sc_wedge(c0, c1, seg2d):
  @functools.partial(
      pl.kernel,
      mesh=_mesh(),
      compiler_params=_sc_params(),
      out_type=jax.ShapeDtypeStruct((_E, _LANES), jnp.float32),
      scratch_types=[
          pltpu.VMEM((2, _K, _LANES), jnp.float32),
          pltpu.VMEM((2, _K, _LANES), jnp.float32),
          pltpu.VMEM((_NCHUNK, _K), jnp.int32),
          pltpu.SemaphoreType.DMA,
          pltpu.SemaphoreType.DMA,
          pltpu.SemaphoreType.DMA,
          pltpu.SemaphoreType.DMA,
      ],
  )
  def k(c0_hbm, c1_hbm, seg_hbm, out_hbm, g0, g1, si, s0a, s1a, s0b, s1b):
    c = lax.axis_index("c")
    s = lax.axis_index("s")
    wid = c * _NS + s
    ebase = wid * _EPW
    sems = ((s0a, s1a), (s0b, s1b))

    pltpu.sync_copy(seg_hbm.at[pl.ds(wid * _NCHUNK, _NCHUNK)], si)

    def issue(ci, b):
      pltpu.async_copy(c0_hbm.at[si.at[ci]], g0.at[b], sems[b][0])
      pltpu.async_copy(c1_hbm.at[si.at[ci]], g1.at[b], sems[b][1])

    def process(ci, b):
      pltpu.make_async_copy(c0_hbm.at[si.at[0]], g0.at[b], sems[b][0]).wait()
      pltpu.make_async_copy(c1_hbm.at[si.at[0]], g1.at[b], sems[b][1]).wait()

      def row(i, carry):
        g0[b, i, :] = 1.0 / jnp.maximum(g0[b, i, :] + g1[b, i, :], 1.0)
        return carry

      lax.fori_loop(0, _K, row, 0)
      pltpu.sync_copy(g0.at[b], out_hbm.at[pl.ds(ebase + ci * _K, _K)])

    issue(0, 0)

    def pair(j, carry):
      issue(2 * j + 1, 1)
      process(2 * j, 0)
      issue(2 * j + 2, 0)
      process(2 * j + 1, 1)
      return carry

    lax.fori_loop(0, (_NCHUNK - 1) // 2, pair, 0)
    process(_NCHUNK - 1, 0)

  return k(c0, c1, seg2d)


def _sc_edge_pass(mm, wedge, gidx2d, dst2d, width):
  nsub = width // _LANES

  @functools.partial(
      pl.kernel,
      mesh=_mesh(),
      compiler_params=_sc_params(),
      out_type=jax.ShapeDtypeStruct((_NC, _NP, width), jnp.float32),
      scratch_types=[
          pltpu.VMEM_SHARED((_NP, width), jnp.float32),
          pltpu.VMEM((3, _K, width), jnp.float32),
          pltpu.VMEM((3, _K, _LANES), jnp.float32),
          pltpu.VMEM((_NCHUNK, _K), jnp.int32),
          pltpu.VMEM((3, _K), jnp.int32),
          [pltpu.SemaphoreType.DMA] * 12,
      ],
  )
  def k(mm_hbm, wedge_hbm, gidx_hbm, dst_hbm, out_hbm,
        acc, feat, wrow, gi, di, sems):
    c = lax.axis_index("c")
    s = lax.axis_index("s")
    wid = c * _NS + s
    ebase = wid * _EPW
    gsem = sems[0:3]
    wsem = sems[3:6]
    ssem = sems[6:9]
    dsem = sems[9:12]
    zv = jnp.zeros((_LANES,), jnp.float32)

    def fillz(i, carry):
      for j in range(nsub):
        feat[0, i, pl.ds(j * _LANES, _LANES)] = zv
        feat[2, i, pl.ds(j * _LANES, _LANES)] = zv
      return carry

    lax.fori_loop(0, _K, fillz, 0)

    pltpu.sync_copy(gidx_hbm.at[pl.ds(wid * _NCHUNK, _NCHUNK)], gi)
    for p in range(_RT // _K):
      pltpu.sync_copy(feat.at[0], acc.at[pl.ds(s * _RT + p * _K, _K)])
    pltpu.sync_copy(dst_hbm.at[wid * _NCHUNK], di.at[2])
    plsc.subcore_barrier()

    def issue(ci, b):
      pltpu.async_copy(mm_hbm.at[gi.at[ci]], feat.at[b], gsem[b])
      pltpu.async_copy(wedge_hbm.at[pl.ds(ebase + ci * _K, _K)], wrow.at[b], wsem[b])
      pltpu.async_copy(dst_hbm.at[wid * _NCHUNK + ci], di.at[b], dsem[b])

    def wait_in(b):
      pltpu.make_async_copy(mm_hbm.at[gi.at[0]], feat.at[b], gsem[b]).wait()
      pltpu.make_async_copy(wedge_hbm.at[pl.ds(ebase, _K)], wrow.at[b], wsem[b]).wait()
      pltpu.make_async_copy(dst_hbm.at[wid * _NCHUNK], di.at[b], dsem[b]).wait()

    def scale(b):
      def row(i, carry):
        w = wrow[b, i, :]
        for j in range(nsub):
          feat[b, i, pl.ds(j * _LANES, _LANES)] = feat[b, i, pl.ds(j * _LANES, _LANES)] * w
        return carry

      lax.fori_loop(0, _K, row, 0)

    def start_scatter(b):
      pltpu.async_copy(feat.at[b], acc.at[di.at[b]], ssem[b], add=True)

    def wait_scatter(b):
      pltpu.make_async_copy(feat.at[b], acc.at[di.at[b]], ssem[b]).wait()

    def phase(b, bnext, nxt):
      wait_in(b)
      scale(b)
      start_scatter(b)
      wait_scatter(bnext)
      if nxt is not None:
        issue(nxt, bnext)

    issue(0, 0)
    issue(1, 1)
    start_scatter(2)

    def triple(j, carry):
      phase(0, 2, 3 * j + 2)
      phase(1, 0, 3 * j + 3)
      phase(2, 1, 3 * j + 4)
      return carry

    lax.fori_loop(0, (_NCHUNK - 2) // 3, triple, 0)
    phase(0, 2, None)
    phase(1, 0, None)
    wait_scatter(1)
    plsc.subcore_barrier()

    for p in range(_RT // _K):
      r0 = s * _RT + p * _K
      pltpu.sync_copy(acc.at[pl.ds(r0, _K)], feat.at[0])
      pltpu.sync_copy(feat.at[0], out_hbm.at[c, pl.ds(r0, _K)])

  return k(mm, wedge, gidx2d, dst2d)


def _tc_transform(h, wstack):
  nine, din, dout = wstack.shape
  bn = 2000
  nb = _N // bn

  def body(h_ref, w_ref, o_ref):
    o_ref[...] = jnp.dot(h_ref[...], w_ref[0], preferred_element_type=jnp.float32)

  return pl.pallas_call(
      body,
      grid=(nine, nb),
      in_specs=[
          pl.BlockSpec((bn, din), lambda j, i: (i, 0)),
          pl.BlockSpec((1, din, dout), lambda j, i: (j, 0, 0)),
      ],
      out_specs=pl.BlockSpec((bn, dout), lambda j, i: (j * nb + i, 0)),
      out_shape=jax.ShapeDtypeStruct((nine * _N, dout), jnp.float32),
  )(h, wstack)


def _tc_fuse(a, mm, b2, relu):
  width = a.shape[-1]
  bn = 2000
  nb = _N // bn

  def body(a0_ref, a1_ref, r_ref, b_ref, o_ref):
    z = a0_ref[0] + a1_ref[0] + r_ref[...] + b_ref[0]
    if relu:
      o_ref[...] = jnp.maximum(z, 0.0)
    else:
      m = jnp.max(z, axis=-1, keepdims=True)
      e = jnp.exp(z - m)
      o_ref[...] = z - m - jnp.log(jnp.sum(e, axis=-1, keepdims=True))

  return pl.pallas_call(
      body,
      grid=(nb,),
      in_specs=[
          pl.BlockSpec((1, bn, width), lambda i: (0, i, 0)),
          pl.BlockSpec((1, bn, width), lambda i: (1, i, 0)),
          pl.BlockSpec((bn, width), lambda i: (8 * nb + i, 0)),
          pl.BlockSpec((1, width), lambda i: (0, 0)),
      ],
      out_specs=pl.BlockSpec((bn, width), lambda i: (i, 0)),
      out_shape=jax.ShapeDtypeStruct((_N, width), jnp.float32),
  )(a, a, mm, b2)


def kernel(x, edge_index, edge_type,
           W1_rel, W1_root, b1,
           W2_rel, W2_root, b2,
           W3_rel, W3_root, b3,
           W4_rel, W4_root, b4):
  src = edge_index[0].astype(jnp.int32)
  dst = edge_index[1].astype(jnp.int32)
  et = edge_type.astype(jnp.int32)
  gidx2d = (et * _N + src).reshape(_E // _K, _K)
  seg2d = (dst * _R + et).reshape(_E // _K, _K)
  dst2d = dst.reshape(_E // _K, _K)

  cnt = _sc_counts(seg2d)
  wedge = _sc_wedge(cnt[0], cnt[1], seg2d)

  h = x.astype(jnp.float32)
  layers = [
      (W1_rel, W1_root, b1, True),
      (W2_rel, W2_root, b2, True),
      (W3_rel, W3_root, b3, True),
      (W4_rel, W4_root, b4, False),
  ]
  for w_rel, w_root, b, relu in layers:
    wstack = jnp.concatenate([w_rel, w_root[None]], axis=0)
    mm = _tc_transform(h, wstack)
    a = _sc_edge_pass(mm, wedge, gidx2d, dst2d, w_rel.shape[-1])
    h = _tc_fuse(a, mm, b.reshape(1, -1), relu)
  return h


# speedup vs baseline: 24.3221x; 1.0254x over previous
import functools

import jax
import jax.numpy as jnp
from jax import lax
from jax.experimental import pallas as pl
from jax.experimental.pallas import tpu as pltpu
from jax.experimental.pallas import tpu_sc as plsc

_N = 10000
_E = 320000
_R = 8
_LANES = 16

_NC = 2
_NS = 16
_NW = _NC * _NS
_EPW = _E // _NW
_K = 80
_NCHUNK = _EPW // _K

_NP = 10240
_RT = _NP // _NS
_ZP = 5
_ZR = _RT // _ZP

_SP = 81920
_RT2 = _SP // _NS
_ZR2 = _RT2 // _ZP

_mesh = lambda: plsc.VectorSubcoreMesh(core_axis_name="c", subcore_axis_name="s")
_sc_params = lambda: pltpu.CompilerParams(use_tc_tiling_on_sc=False)


def _sc_counts(seg2d):
  @functools.partial(
      pl.kernel,
      mesh=_mesh(),
      compiler_params=_sc_params(),
      out_type=jax.ShapeDtypeStruct((_NC, _SP, _LANES), jnp.float32),
      scratch_types=[
          pltpu.VMEM_SHARED((_SP, _LANES), jnp.float32),
          pltpu.VMEM((_ZR2, _LANES), jnp.float32),
          pltpu.VMEM((_K, _LANES), jnp.float32),
          pltpu.VMEM((_NCHUNK, _K), jnp.int32),
          pltpu.SemaphoreType.DMA,
      ],
  )
  def k(seg_hbm, out_hbm, acc, zb, ones, si, sem):
    c = lax.axis_index("c")
    s = lax.axis_index("s")
    wid = c * _NS + s
    zv = jnp.zeros((_LANES,), jnp.float32)
    ov = jnp.ones((_LANES,), jnp.float32)

    def fill(i, carry):
      zb[i, :] = zv
      return carry

    lax.fori_loop(0, _ZR2, fill, 0)

    def fillo(i, carry):
      ones[i, :] = ov
      return carry

    lax.fori_loop(0, _K, fillo, 0)

    pltpu.sync_copy(seg_hbm.at[pl.ds(wid * _NCHUNK, _NCHUNK)], si)
    for p in range(_ZP):
      pltpu.sync_copy(zb, acc.at[pl.ds(s * _RT2 + p * _ZR2, _ZR2)])
    plsc.subcore_barrier()

    def group(g, carry):
      for i in range(25):
        pltpu.async_copy(ones, acc.at[si.at[g * 25 + i]], sem, add=True)
      for i in range(25):
        pltpu.make_async_copy(ones, acc.at[si.at[0]], sem).wait()
      return carry

    lax.fori_loop(0, _NCHUNK // 25, group, 0)
    plsc.subcore_barrier()

    for p in range(_ZP):
      r0 = s * _RT2 + p * _ZR2
      pltpu.sync_copy(acc.at[pl.ds(r0, _ZR2)], zb)
      pltpu.sync_copy(zb, out_hbm.at[c, pl.ds(r0, _ZR2)])

  return k(seg2d)


def _sc_wedge(c0, c1, seg2d):
  @functools.partial(
      pl.kernel,
      mesh=_mesh(),
      compiler_params=_sc_params(),
      out_type=jax.ShapeDtypeStruct((_E, _LANES), jnp.float32),
      scratch_types=[
          pltpu.VMEM((2, _K, _LANES), jnp.float32),
          pltpu.VMEM((2, _K, _LANES), jnp.float32),
          pltpu.VMEM((_NCHUNK, _K), jnp.int32),
          pltpu.SemaphoreType.DMA,
          pltpu.SemaphoreType.DMA,
          pltpu.SemaphoreType.DMA,
          pltpu.SemaphoreType.DMA,
      ],
  )
  def k(c0_hbm, c1_hbm, seg_hbm, out_hbm, g0, g1, si, s0a, s1a, s0b, s1b):
    c = lax.axis_index("c")
    s = lax.axis_index("s")
    wid = c * _NS + s
    ebase = wid * _EPW
    sems = ((s0a, s1a), (s0b, s1b))

    pltpu.sync_copy(seg_hbm.at[pl.ds(wid * _NCHUNK, _NCHUNK)], si)

    def issue(ci, b):
      pltpu.async_copy(c0_hbm.at[si.at[ci]], g0.at[b], sems[b][0])
      pltpu.async_copy(c1_hbm.at[si.at[ci]], g1.at[b], sems[b][1])

    def process(ci, b):
      pltpu.make_async_copy(c0_hbm.at[si.at[0]], g0.at[b], sems[b][0]).wait()
      pltpu.make_async_copy(c1_hbm.at[si.at[0]], g1.at[b], sems[b][1]).wait()

      def row(i, carry):
        g0[b, i, :] = 1.0 / jnp.maximum(g0[b, i, :] + g1[b, i, :], 1.0)
        return carry

      lax.fori_loop(0, _K, row, 0)
      pltpu.sync_copy(g0.at[b], out_hbm.at[pl.ds(ebase + ci * _K, _K)])

    issue(0, 0)

    def pair(j, carry):
      issue(2 * j + 1, 1)
      process(2 * j, 0)
      issue(2 * j + 2, 0)
      process(2 * j + 1, 1)
      return carry

    lax.fori_loop(0, (_NCHUNK - 1) // 2, pair, 0)
    process(_NCHUNK - 1, 0)

  return k(c0, c1, seg2d)


def _sc_edge_pass(mm, wedge, gidx2d, dst2d, width):
  nsub = width // _LANES

  @functools.partial(
      pl.kernel,
      mesh=_mesh(),
      compiler_params=_sc_params(),
      out_type=jax.ShapeDtypeStruct((_NC, _NP, width), jnp.float32),
      scratch_types=[
          pltpu.VMEM_SHARED((_NP, width), jnp.float32),
          pltpu.VMEM((3, _K, width), jnp.float32),
          pltpu.VMEM((3, _K, _LANES), jnp.float32),
          pltpu.VMEM((_NCHUNK, _K), jnp.int32),
          pltpu.VMEM((3, _K), jnp.int32),
          [pltpu.SemaphoreType.DMA] * 12,
      ],
  )
  def k(mm_hbm, wedge_hbm, gidx_hbm, dst_hbm, out_hbm,
        acc, feat, wrow, gi, di, sems):
    c = lax.axis_index("c")
    s = lax.axis_index("s")
    wid = c * _NS + s
    ebase = wid * _EPW
    gsem = sems[0:3]
    wsem = sems[3:6]
    ssem = sems[6:9]
    dsem = sems[9:12]
    zv = jnp.zeros((_LANES,), jnp.float32)

    def fillz(i, carry):
      for j in range(nsub):
        feat[0, i, pl.ds(j * _LANES, _LANES)] = zv
        feat[2, i, pl.ds(j * _LANES, _LANES)] = zv
      return carry

    lax.fori_loop(0, _K, fillz, 0)

    pltpu.sync_copy(gidx_hbm.at[pl.ds(wid * _NCHUNK, _NCHUNK)], gi)
    for p in range(_RT // _K):
      pltpu.sync_copy(feat.at[0], acc.at[pl.ds(s * _RT + p * _K, _K)])
    pltpu.sync_copy(dst_hbm.at[wid * _NCHUNK], di.at[2])
    plsc.subcore_barrier()

    def issue(ci, b):
      pltpu.async_copy(mm_hbm.at[gi.at[ci]], feat.at[b], gsem[b])
      pltpu.async_copy(wedge_hbm.at[pl.ds(ebase + ci * _K, _K)], wrow.at[b], wsem[b])
      pltpu.async_copy(dst_hbm.at[wid * _NCHUNK + ci], di.at[b], dsem[b])

    def wait_in(b):
      pltpu.make_async_copy(mm_hbm.at[gi.at[0]], feat.at[b], gsem[b]).wait()
      pltpu.make_async_copy(wedge_hbm.at[pl.ds(ebase, _K)], wrow.at[b], wsem[b]).wait()
      pltpu.make_async_copy(dst_hbm.at[wid * _NCHUNK], di.at[b], dsem[b]).wait()

    def scale(b):
      def row(i, carry):
        w0 = wrow[b, 2 * i, :]
        w1 = wrow[b, 2 * i + 1, :]
        for j in range(nsub):
          feat[b, 2 * i, pl.ds(j * _LANES, _LANES)] = feat[b, 2 * i, pl.ds(j * _LANES, _LANES)] * w0
          feat[b, 2 * i + 1, pl.ds(j * _LANES, _LANES)] = feat[b, 2 * i + 1, pl.ds(j * _LANES, _LANES)] * w1
        return carry

      lax.fori_loop(0, _K // 2, row, 0)

    def start_scatter(b):
      pltpu.async_copy(feat.at[b], acc.at[di.at[b]], ssem[b], add=True)

    def wait_scatter(b):
      pltpu.make_async_copy(feat.at[b], acc.at[di.at[b]], ssem[b]).wait()

    def phase(b, bnext, nxt):
      wait_in(b)
      scale(b)
      start_scatter(b)
      wait_scatter(bnext)
      if nxt is not None:
        issue(nxt, bnext)

    issue(0, 0)
    issue(1, 1)
    start_scatter(2)

    def triple(j, carry):
      phase(0, 2, 3 * j + 2)
      phase(1, 0, 3 * j + 3)
      phase(2, 1, 3 * j + 4)
      return carry

    lax.fori_loop(0, (_NCHUNK - 2) // 3, triple, 0)
    phase(0, 2, None)
    phase(1, 0, None)
    wait_scatter(1)
    plsc.subcore_barrier()

    for p in range(_RT // _K):
      r0 = s * _RT + p * _K
      pltpu.sync_copy(acc.at[pl.ds(r0, _K)], feat.at[0])
      pltpu.sync_copy(feat.at[0], out_hbm.at[c, pl.ds(r0, _K)])

  return k(mm, wedge, gidx2d, dst2d)


def _tc_transform(h, wstack):
  nine, din, dout = wstack.shape
  bn = 2000
  nb = _N // bn

  def body(h_ref, w_ref, o_ref):
    o_ref[...] = jnp.dot(h_ref[...], w_ref[0], preferred_element_type=jnp.float32)

  return pl.pallas_call(
      body,
      grid=(nine, nb),
      in_specs=[
          pl.BlockSpec((bn, din), lambda j, i: (i, 0)),
          pl.BlockSpec((1, din, dout), lambda j, i: (j, 0, 0)),
      ],
      out_specs=pl.BlockSpec((bn, dout), lambda j, i: (j * nb + i, 0)),
      out_shape=jax.ShapeDtypeStruct((nine * _N, dout), jnp.float32),
  )(h, wstack)


def _tc_fuse(a, mm, b2, relu):
  width = a.shape[-1]
  bn = 2000
  nb = _N // bn

  def body(a0_ref, a1_ref, r_ref, b_ref, o_ref):
    z = a0_ref[0] + a1_ref[0] + r_ref[...] + b_ref[0]
    if relu:
      o_ref[...] = jnp.maximum(z, 0.0)
    else:
      m = jnp.max(z, axis=-1, keepdims=True)
      e = jnp.exp(z - m)
      o_ref[...] = z - m - jnp.log(jnp.sum(e, axis=-1, keepdims=True))

  return pl.pallas_call(
      body,
      grid=(nb,),
      in_specs=[
          pl.BlockSpec((1, bn, width), lambda i: (0, i, 0)),
          pl.BlockSpec((1, bn, width), lambda i: (1, i, 0)),
          pl.BlockSpec((bn, width), lambda i: (8 * nb + i, 0)),
          pl.BlockSpec((1, width), lambda i: (0, 0)),
      ],
      out_specs=pl.BlockSpec((bn, width), lambda i: (i, 0)),
      out_shape=jax.ShapeDtypeStruct((_N, width), jnp.float32),
  )(a, a, mm, b2)


def kernel(x, edge_index, edge_type,
           W1_rel, W1_root, b1,
           W2_rel, W2_root, b2,
           W3_rel, W3_root, b3,
           W4_rel, W4_root, b4):
  src = edge_index[0].astype(jnp.int32)
  dst = edge_index[1].astype(jnp.int32)
  et = edge_type.astype(jnp.int32)
  gidx2d = (et * _N + src).reshape(_E // _K, _K)
  seg2d = (dst * _R + et).reshape(_E // _K, _K)
  dst2d = dst.reshape(_E // _K, _K)

  cnt = _sc_counts(seg2d)
  wedge = _sc_wedge(cnt[0], cnt[1], seg2d)

  h = x.astype(jnp.float32)
  layers = [
      (W1_rel, W1_root, b1, True),
      (W2_rel, W2_root, b2, True),
      (W3_rel, W3_root, b3, True),
      (W4_rel, W4_root, b4, False),
  ]
  for w_rel, w_root, b, relu in layers:
    wstack = jnp.concatenate([w_rel, w_root[None]], axis=0)
    mm = _tc_transform(h, wstack)
    a = _sc_edge_pass(mm, wedge, gidx2d, dst2d, w_rel.shape[-1])
    h = _tc_fuse(a, mm, b.reshape(1, -1), relu)
  return h


# speedup vs baseline: 25.4368x; 1.0458x over previous
import functools

import jax
import jax.numpy as jnp
from jax import lax
from jax.experimental import pallas as pl
from jax.experimental.pallas import tpu as pltpu
from jax.experimental.pallas import tpu_sc as plsc

_N = 10000
_E = 320000
_R = 8
_LANES = 16

_NC = 2
_NS = 16
_NW = _NC * _NS
_EPW = _E // _NW
_K = 80
_NCHUNK = _EPW // _K

_NP = 10240
_RT = _NP // _NS
_ZP = 5
_ZR = _RT // _ZP

_SP = 81920
_RT2 = _SP // _NS
_ZR2 = _RT2 // _ZP

_mesh = lambda: plsc.VectorSubcoreMesh(core_axis_name="c", subcore_axis_name="s")
_sc_params = lambda: pltpu.CompilerParams(use_tc_tiling_on_sc=False)


def _sc_counts(seg2d):
  @functools.partial(
      pl.kernel,
      mesh=_mesh(),
      compiler_params=_sc_params(),
      out_type=jax.ShapeDtypeStruct((_NC, _SP, _LANES), jnp.float32),
      scratch_types=[
          pltpu.VMEM_SHARED((_SP, _LANES), jnp.float32),
          pltpu.VMEM((_ZR2, _LANES), jnp.float32),
          pltpu.VMEM((_K, _LANES), jnp.float32),
          pltpu.VMEM((_NCHUNK, _K), jnp.int32),
          pltpu.SemaphoreType.DMA,
      ],
  )
  def k(seg_hbm, out_hbm, acc, zb, ones, si, sem):
    c = lax.axis_index("c")
    s = lax.axis_index("s")
    wid = c * _NS + s
    zv = jnp.zeros((_LANES,), jnp.float32)
    ov = jnp.ones((_LANES,), jnp.float32)

    def fill(i, carry):
      zb[i, :] = zv
      return carry

    lax.fori_loop(0, _ZR2, fill, 0)

    def fillo(i, carry):
      ones[i, :] = ov
      return carry

    lax.fori_loop(0, _K, fillo, 0)

    pltpu.sync_copy(seg_hbm.at[pl.ds(wid * _NCHUNK, _NCHUNK)], si)
    for p in range(_ZP):
      pltpu.sync_copy(zb, acc.at[pl.ds(s * _RT2 + p * _ZR2, _ZR2)])
    plsc.subcore_barrier()

    def group(g, carry):
      for i in range(25):
        pltpu.async_copy(ones, acc.at[si.at[g * 25 + i]], sem, add=True)
      for i in range(25):
        pltpu.make_async_copy(ones, acc.at[si.at[0]], sem).wait()
      return carry

    lax.fori_loop(0, _NCHUNK // 25, group, 0)
    plsc.subcore_barrier()

    for p in range(_ZP):
      r0 = s * _RT2 + p * _ZR2
      pltpu.sync_copy(acc.at[pl.ds(r0, _ZR2)], zb)
      pltpu.sync_copy(zb, out_hbm.at[c, pl.ds(r0, _ZR2)])

  return k(seg2d)


def _sc_wedge(c0, c1, seg2d):
  @functools.partial(
      pl.kernel,
      mesh=_mesh(),
      compiler_params=_sc_params(),
      out_type=jax.ShapeDtypeStruct((_E, _LANES), jnp.float32),
      scratch_types=[
          pltpu.VMEM((2, _K, _LANES), jnp.float32),
          pltpu.VMEM((2, _K, _LANES), jnp.float32),
          pltpu.VMEM((_NCHUNK, _K), jnp.int32),
          pltpu.SemaphoreType.DMA,
          pltpu.SemaphoreType.DMA,
          pltpu.SemaphoreType.DMA,
          pltpu.SemaphoreType.DMA,
      ],
  )
  def k(c0_hbm, c1_hbm, seg_hbm, out_hbm, g0, g1, si, s0a, s1a, s0b, s1b):
    c = lax.axis_index("c")
    s = lax.axis_index("s")
    wid = c * _NS + s
    ebase = wid * _EPW
    sems = ((s0a, s1a), (s0b, s1b))

    pltpu.sync_copy(seg_hbm.at[pl.ds(wid * _NCHUNK, _NCHUNK)], si)

    def issue(ci, b):
      pltpu.async_copy(c0_hbm.at[si.at[ci]], g0.at[b], sems[b][0])
      pltpu.async_copy(c1_hbm.at[si.at[ci]], g1.at[b], sems[b][1])

    def process(ci, b):
      pltpu.make_async_copy(c0_hbm.at[si.at[0]], g0.at[b], sems[b][0]).wait()
      pltpu.make_async_copy(c1_hbm.at[si.at[0]], g1.at[b], sems[b][1]).wait()

      def row(i, carry):
        g0[b, i, :] = 1.0 / jnp.maximum(g0[b, i, :] + g1[b, i, :], 1.0)
        return carry

      lax.fori_loop(0, _K, row, 0)
      pltpu.sync_copy(g0.at[b], out_hbm.at[pl.ds(ebase + ci * _K, _K)])

    issue(0, 0)

    def pair(j, carry):
      issue(2 * j + 1, 1)
      process(2 * j, 0)
      issue(2 * j + 2, 0)
      process(2 * j + 1, 1)
      return carry

    lax.fori_loop(0, (_NCHUNK - 1) // 2, pair, 0)
    process(_NCHUNK - 1, 0)

  return k(c0, c1, seg2d)


def _sc_edge_pass(mm, wedge, gidx2d, dst2d, width):
  nsub = width // _LANES

  @functools.partial(
      pl.kernel,
      mesh=_mesh(),
      compiler_params=_sc_params(),
      out_type=jax.ShapeDtypeStruct((_NC, _NP, width), jnp.float32),
      scratch_types=[
          pltpu.VMEM_SHARED((_NP, width), jnp.float32),
          pltpu.VMEM((3, _K, width), jnp.float32),
          pltpu.VMEM((3, _K, _LANES), jnp.float32),
          pltpu.VMEM((_NCHUNK, _K), jnp.int32),
          pltpu.VMEM((3, _K), jnp.int32),
          [pltpu.SemaphoreType.DMA] * 12,
      ],
  )
  def k(mm_hbm, wedge_hbm, gidx_hbm, dst_hbm, out_hbm,
        acc, feat, wrow, gi, di, sems):
    c = lax.axis_index("c")
    s = lax.axis_index("s")
    wid = c * _NS + s
    ebase = wid * _EPW
    gsem = sems[0:3]
    wsem = sems[3:6]
    ssem = sems[6:9]
    dsem = sems[9:12]
    zv = jnp.zeros((_LANES,), jnp.float32)

    def fillz(i, carry):
      for j in range(nsub):
        feat[0, i, pl.ds(j * _LANES, _LANES)] = zv
        feat[2, i, pl.ds(j * _LANES, _LANES)] = zv
      return carry

    lax.fori_loop(0, _K, fillz, 0)

    pltpu.sync_copy(gidx_hbm.at[pl.ds(wid * _NCHUNK, _NCHUNK)], gi)
    for p in range(_RT // _K):
      pltpu.sync_copy(feat.at[0], acc.at[pl.ds(s * _RT + p * _K, _K)])
    pltpu.sync_copy(dst_hbm.at[wid * _NCHUNK], di.at[2])
    plsc.subcore_barrier()

    def issue(ci, b):
      pltpu.async_copy(mm_hbm.at[gi.at[ci]], feat.at[b], gsem[b])
      pltpu.async_copy(wedge_hbm.at[pl.ds(ebase + ci * _K, _K)], wrow.at[b], wsem[b])
      pltpu.async_copy(dst_hbm.at[wid * _NCHUNK + ci], di.at[b], dsem[b])

    def wait_in(b):
      pltpu.make_async_copy(mm_hbm.at[gi.at[0]], feat.at[b], gsem[b]).wait()
      pltpu.make_async_copy(wedge_hbm.at[pl.ds(ebase, _K)], wrow.at[b], wsem[b]).wait()
      pltpu.make_async_copy(dst_hbm.at[wid * _NCHUNK], di.at[b], dsem[b]).wait()

    def scale(b):
      def row(i, carry):
        w0 = wrow[b, 2 * i, :]
        w1 = wrow[b, 2 * i + 1, :]
        for j in range(nsub):
          feat[b, 2 * i, pl.ds(j * _LANES, _LANES)] = feat[b, 2 * i, pl.ds(j * _LANES, _LANES)] * w0
          feat[b, 2 * i + 1, pl.ds(j * _LANES, _LANES)] = feat[b, 2 * i + 1, pl.ds(j * _LANES, _LANES)] * w1
        return carry

      lax.fori_loop(0, _K // 2, row, 0)

    def start_scatter(b):
      pltpu.async_copy(feat.at[b], acc.at[di.at[b]], ssem[b], add=True)

    def wait_scatter(b):
      pltpu.make_async_copy(feat.at[b], acc.at[di.at[b]], ssem[b]).wait()

    def phase(b, bnext, nxt):
      wait_in(b)
      scale(b)
      start_scatter(b)
      wait_scatter(bnext)
      if nxt is not None:
        issue(nxt, bnext)

    issue(0, 0)
    issue(1, 1)
    start_scatter(2)

    def triple(j, carry):
      phase(0, 2, 3 * j + 2)
      phase(1, 0, 3 * j + 3)
      phase(2, 1, 3 * j + 4)
      return carry

    lax.fori_loop(0, (_NCHUNK - 2) // 3, triple, 0)
    phase(0, 2, None)
    phase(1, 0, None)
    wait_scatter(1)
    plsc.subcore_barrier()

    for p in range(_RT // _K):
      r0 = s * _RT + p * _K
      pltpu.sync_copy(acc.at[pl.ds(r0, _K)], feat.at[0])
      pltpu.sync_copy(feat.at[0], out_hbm.at[c, pl.ds(r0, _K)])

  return k(mm, wedge, gidx2d, dst2d)


def _tc_transform(h, wstack):
  nine, din, dout = wstack.shape
  bn = 2000
  nb = _N // bn

  def body(h_ref, w_ref, o_ref):
    o_ref[...] = jnp.dot(h_ref[...], w_ref[0], preferred_element_type=jnp.float32)

  return pl.pallas_call(
      body,
      grid=(nine, nb),
      in_specs=[
          pl.BlockSpec((bn, din), lambda j, i: (i, 0)),
          pl.BlockSpec((1, din, dout), lambda j, i: (j, 0, 0)),
      ],
      out_specs=pl.BlockSpec((bn, dout), lambda j, i: (j * nb + i, 0)),
      out_shape=jax.ShapeDtypeStruct((nine * _N, dout), jnp.float32),
  )(h, wstack)


def _tc_transform_fused(a, mm_prev, b2, wstack):
  nine, din, dout = wstack.shape
  bn = 2000
  nb = _N // bn

  def body(a0_ref, a1_ref, r_ref, b_ref, w_ref, o_ref, hbuf):
    @pl.when(pl.program_id(1) == 0)
    def _():
      hbuf[...] = jnp.maximum(a0_ref[0] + a1_ref[0] + r_ref[...] + b_ref[0], 0.0)

    o_ref[...] = jnp.dot(hbuf[...], w_ref[0], preferred_element_type=jnp.float32)

  return pl.pallas_call(
      body,
      grid=(nb, nine),
      in_specs=[
          pl.BlockSpec((1, bn, din), lambda i, j: (0, i, 0)),
          pl.BlockSpec((1, bn, din), lambda i, j: (1, i, 0)),
          pl.BlockSpec((bn, din), lambda i, j: (8 * nb + i, 0)),
          pl.BlockSpec((1, din), lambda i, j: (0, 0)),
          pl.BlockSpec((1, din, dout), lambda i, j: (j, 0, 0)),
      ],
      out_specs=pl.BlockSpec((bn, dout), lambda i, j: (j * nb + i, 0)),
      out_shape=jax.ShapeDtypeStruct((nine * _N, dout), jnp.float32),
      scratch_shapes=[pltpu.VMEM((bn, din), jnp.float32)],
  )(a, a, mm_prev, b2, wstack)


def _tc_fuse(a, mm, b2, relu):
  width = a.shape[-1]
  bn = 2000
  nb = _N // bn

  def body(a0_ref, a1_ref, r_ref, b_ref, o_ref):
    z = a0_ref[0] + a1_ref[0] + r_ref[...] + b_ref[0]
    if relu:
      o_ref[...] = jnp.maximum(z, 0.0)
    else:
      m = jnp.max(z, axis=-1, keepdims=True)
      e = jnp.exp(z - m)
      o_ref[...] = z - m - jnp.log(jnp.sum(e, axis=-1, keepdims=True))

  return pl.pallas_call(
      body,
      grid=(nb,),
      in_specs=[
          pl.BlockSpec((1, bn, width), lambda i: (0, i, 0)),
          pl.BlockSpec((1, bn, width), lambda i: (1, i, 0)),
          pl.BlockSpec((bn, width), lambda i: (8 * nb + i, 0)),
          pl.BlockSpec((1, width), lambda i: (0, 0)),
      ],
      out_specs=pl.BlockSpec((bn, width), lambda i: (i, 0)),
      out_shape=jax.ShapeDtypeStruct((_N, width), jnp.float32),
  )(a, a, mm, b2)


def kernel(x, edge_index, edge_type,
           W1_rel, W1_root, b1,
           W2_rel, W2_root, b2,
           W3_rel, W3_root, b3,
           W4_rel, W4_root, b4):
  src = edge_index[0].astype(jnp.int32)
  dst = edge_index[1].astype(jnp.int32)
  et = edge_type.astype(jnp.int32)
  gidx2d = (et * _N + src).reshape(_E // _K, _K)
  seg2d = (dst * _R + et).reshape(_E // _K, _K)
  dst2d = dst.reshape(_E // _K, _K)

  cnt = _sc_counts(seg2d)
  wedge = _sc_wedge(cnt[0], cnt[1], seg2d)

  stacks = [jnp.concatenate([w_rel, w_root[None]], axis=0)
            for w_rel, w_root in ((W1_rel, W1_root), (W2_rel, W2_root),
                                  (W3_rel, W3_root), (W4_rel, W4_root))]
  mm = _tc_transform(x.astype(jnp.float32), stacks[0])
  a = _sc_edge_pass(mm, wedge, gidx2d, dst2d, 128)
  for li, b in ((1, b1), (2, b2), (3, b3)):
    mm = _tc_transform_fused(a, mm, b.reshape(1, -1), stacks[li])
    a = _sc_edge_pass(mm, wedge, gidx2d, dst2d, stacks[li].shape[-1])
  return _tc_fuse(a, mm, b4.reshape(1, -1), relu=False)


# speedup vs baseline: 25.5398x; 1.0040x over previous
import functools

import jax
import jax.numpy as jnp
from jax import lax
from jax.experimental import pallas as pl
from jax.experimental.pallas import tpu as pltpu
from jax.experimental.pallas import tpu_sc as plsc

_N = 10000
_E = 320000
_R = 8
_LANES = 16

_NC = 2
_NS = 16
_NW = _NC * _NS
_EPW = _E // _NW
_K = 80
_NCHUNK = _EPW // _K

_NP = 10240
_RT = _NP // _NS
_ZP = 5
_ZR = _RT // _ZP

_SP = 81920
_RT2 = _SP // _NS
_ZR2 = _RT2 // _ZP

_mesh = lambda: plsc.VectorSubcoreMesh(core_axis_name="c", subcore_axis_name="s")
_sc_params = lambda: pltpu.CompilerParams(use_tc_tiling_on_sc=False)


def _sc_counts(seg2d):
  @functools.partial(
      pl.kernel,
      mesh=_mesh(),
      compiler_params=_sc_params(),
      out_type=jax.ShapeDtypeStruct((_NC, _SP, _LANES), jnp.float32),
      scratch_types=[
          pltpu.VMEM_SHARED((_SP, _LANES), jnp.float32),
          pltpu.VMEM((_ZR2, _LANES), jnp.float32),
          pltpu.VMEM((_K, _LANES), jnp.float32),
          pltpu.VMEM((_NCHUNK, _K), jnp.int32),
          pltpu.SemaphoreType.DMA,
      ],
  )
  def k(seg_hbm, out_hbm, acc, zb, ones, si, sem):
    c = lax.axis_index("c")
    s = lax.axis_index("s")
    wid = c * _NS + s
    zv = jnp.zeros((_LANES,), jnp.float32)
    ov = jnp.ones((_LANES,), jnp.float32)

    def fill(i, carry):
      zb[i, :] = zv
      return carry

    lax.fori_loop(0, _ZR2, fill, 0)

    def fillo(i, carry):
      ones[i, :] = ov
      return carry

    lax.fori_loop(0, _K, fillo, 0)

    pltpu.sync_copy(seg_hbm.at[pl.ds(wid * _NCHUNK, _NCHUNK)], si)
    for p in range(_ZP):
      pltpu.sync_copy(zb, acc.at[pl.ds(s * _RT2 + p * _ZR2, _ZR2)])
    plsc.subcore_barrier()

    def group(g, carry):
      for i in range(25):
        pltpu.async_copy(ones, acc.at[si.at[g * 25 + i]], sem, add=True)
      for i in range(25):
        pltpu.make_async_copy(ones, acc.at[si.at[0]], sem).wait()
      return carry

    lax.fori_loop(0, _NCHUNK // 25, group, 0)
    plsc.subcore_barrier()

    pltpu.sync_copy(acc.at[pl.ds(s * _RT2, _RT2)],
                    out_hbm.at[c, pl.ds(s * _RT2, _RT2)])

  return k(seg2d)


def _sc_wedge(c0, c1, seg2d):
  @functools.partial(
      pl.kernel,
      mesh=_mesh(),
      compiler_params=_sc_params(),
      out_type=jax.ShapeDtypeStruct((_E, _LANES), jnp.float32),
      scratch_types=[
          pltpu.VMEM((2, _K, _LANES), jnp.float32),
          pltpu.VMEM((2, _K, _LANES), jnp.float32),
          pltpu.VMEM((_NCHUNK, _K), jnp.int32),
          pltpu.SemaphoreType.DMA,
          pltpu.SemaphoreType.DMA,
          pltpu.SemaphoreType.DMA,
          pltpu.SemaphoreType.DMA,
      ],
  )
  def k(c0_hbm, c1_hbm, seg_hbm, out_hbm, g0, g1, si, s0a, s1a, s0b, s1b):
    c = lax.axis_index("c")
    s = lax.axis_index("s")
    wid = c * _NS + s
    ebase = wid * _EPW
    sems = ((s0a, s1a), (s0b, s1b))

    pltpu.sync_copy(seg_hbm.at[pl.ds(wid * _NCHUNK, _NCHUNK)], si)

    def issue(ci, b):
      pltpu.async_copy(c0_hbm.at[si.at[ci]], g0.at[b], sems[b][0])
      pltpu.async_copy(c1_hbm.at[si.at[ci]], g1.at[b], sems[b][1])

    def process(ci, b):
      pltpu.make_async_copy(c0_hbm.at[si.at[0]], g0.at[b], sems[b][0]).wait()
      pltpu.make_async_copy(c1_hbm.at[si.at[0]], g1.at[b], sems[b][1]).wait()

      def row(i, carry):
        g0[b, i, :] = 1.0 / jnp.maximum(g0[b, i, :] + g1[b, i, :], 1.0)
        return carry

      lax.fori_loop(0, _K, row, 0)
      pltpu.sync_copy(g0.at[b], out_hbm.at[pl.ds(ebase + ci * _K, _K)])

    issue(0, 0)

    def pair(j, carry):
      issue(2 * j + 1, 1)
      process(2 * j, 0)
      issue(2 * j + 2, 0)
      process(2 * j + 1, 1)
      return carry

    lax.fori_loop(0, (_NCHUNK - 1) // 2, pair, 0)
    process(_NCHUNK - 1, 0)

  return k(c0, c1, seg2d)


def _sc_edge_pass(mm, wedge, gidx2d, dst2d, width):
  nsub = width // _LANES

  @functools.partial(
      pl.kernel,
      mesh=_mesh(),
      compiler_params=_sc_params(),
      out_type=jax.ShapeDtypeStruct((_NC, _NP, width), jnp.float32),
      scratch_types=[
          pltpu.VMEM_SHARED((_NP, width), jnp.float32),
          pltpu.VMEM((3, _K, width), jnp.float32),
          pltpu.VMEM((3, _K, _LANES), jnp.float32),
          pltpu.VMEM((_NCHUNK, _K), jnp.int32),
          pltpu.VMEM((3, _K), jnp.int32),
          [pltpu.SemaphoreType.DMA] * 12,
      ],
  )
  def k(mm_hbm, wedge_hbm, gidx_hbm, dst_hbm, out_hbm,
        acc, feat, wrow, gi, di, sems):
    c = lax.axis_index("c")
    s = lax.axis_index("s")
    wid = c * _NS + s
    ebase = wid * _EPW
    gsem = sems[0:3]
    wsem = sems[3:6]
    ssem = sems[6:9]
    dsem = sems[9:12]
    zv = jnp.zeros((_LANES,), jnp.float32)

    def fillz(i, carry):
      for j in range(nsub):
        feat[0, i, pl.ds(j * _LANES, _LANES)] = zv
        feat[2, i, pl.ds(j * _LANES, _LANES)] = zv
      return carry

    lax.fori_loop(0, _K, fillz, 0)

    pltpu.sync_copy(gidx_hbm.at[pl.ds(wid * _NCHUNK, _NCHUNK)], gi)
    for p in range(_RT // _K):
      pltpu.sync_copy(feat.at[0], acc.at[pl.ds(s * _RT + p * _K, _K)])
    pltpu.sync_copy(dst_hbm.at[wid * _NCHUNK], di.at[2])
    plsc.subcore_barrier()

    def issue(ci, b):
      pltpu.async_copy(mm_hbm.at[gi.at[ci]], feat.at[b], gsem[b])
      pltpu.async_copy(wedge_hbm.at[pl.ds(ebase + ci * _K, _K)], wrow.at[b], wsem[b])
      pltpu.async_copy(dst_hbm.at[wid * _NCHUNK + ci], di.at[b], dsem[b])

    def wait_in(b):
      pltpu.make_async_copy(mm_hbm.at[gi.at[0]], feat.at[b], gsem[b]).wait()
      pltpu.make_async_copy(wedge_hbm.at[pl.ds(ebase, _K)], wrow.at[b], wsem[b]).wait()
      pltpu.make_async_copy(dst_hbm.at[wid * _NCHUNK], di.at[b], dsem[b]).wait()

    def scale(b):
      def row(i, carry):
        w0 = wrow[b, 2 * i, :]
        w1 = wrow[b, 2 * i + 1, :]
        for j in range(nsub):
          feat[b, 2 * i, pl.ds(j * _LANES, _LANES)] = feat[b, 2 * i, pl.ds(j * _LANES, _LANES)] * w0
          feat[b, 2 * i + 1, pl.ds(j * _LANES, _LANES)] = feat[b, 2 * i + 1, pl.ds(j * _LANES, _LANES)] * w1
        return carry

      lax.fori_loop(0, _K // 2, row, 0)

    def start_scatter(b):
      pltpu.async_copy(feat.at[b], acc.at[di.at[b]], ssem[b], add=True)

    def wait_scatter(b):
      pltpu.make_async_copy(feat.at[b], acc.at[di.at[b]], ssem[b]).wait()

    def phase(b, bnext, nxt):
      wait_in(b)
      scale(b)
      start_scatter(b)
      wait_scatter(bnext)
      if nxt is not None:
        issue(nxt, bnext)

    issue(0, 0)
    issue(1, 1)
    start_scatter(2)

    def triple(j, carry):
      phase(0, 2, 3 * j + 2)
      phase(1, 0, 3 * j + 3)
      phase(2, 1, 3 * j + 4)
      return carry

    lax.fori_loop(0, (_NCHUNK - 2) // 3, triple, 0)
    phase(0, 2, None)
    phase(1, 0, None)
    wait_scatter(1)
    plsc.subcore_barrier()

    pltpu.sync_copy(acc.at[pl.ds(s * _RT, _RT)], out_hbm.at[c, pl.ds(s * _RT, _RT)])

  return k(mm, wedge, gidx2d, dst2d)


def _tc_transform(h, wstack):
  nine, din, dout = wstack.shape
  bn = 2000
  nb = _N // bn

  def body(h_ref, w_ref, o_ref):
    o_ref[...] = jnp.dot(h_ref[...], w_ref[0], preferred_element_type=jnp.float32)

  return pl.pallas_call(
      body,
      grid=(nine, nb),
      in_specs=[
          pl.BlockSpec((bn, din), lambda j, i: (i, 0)),
          pl.BlockSpec((1, din, dout), lambda j, i: (j, 0, 0)),
      ],
      out_specs=pl.BlockSpec((bn, dout), lambda j, i: (j * nb + i, 0)),
      out_shape=jax.ShapeDtypeStruct((nine * _N, dout), jnp.float32),
  )(h, wstack)


def _tc_transform_fused(a, mm_prev, b2, wstack):
  nine, din, dout = wstack.shape
  bn = 2000
  nb = _N // bn

  def body(a0_ref, a1_ref, r_ref, b_ref, w_ref, o_ref, hbuf):
    @pl.when(pl.program_id(1) == 0)
    def _():
      hbuf[...] = jnp.maximum(a0_ref[0] + a1_ref[0] + r_ref[...] + b_ref[0], 0.0)

    o_ref[...] = jnp.dot(hbuf[...], w_ref[0], preferred_element_type=jnp.float32)

  return pl.pallas_call(
      body,
      grid=(nb, nine),
      in_specs=[
          pl.BlockSpec((1, bn, din), lambda i, j: (0, i, 0)),
          pl.BlockSpec((1, bn, din), lambda i, j: (1, i, 0)),
          pl.BlockSpec((bn, din), lambda i, j: (8 * nb + i, 0)),
          pl.BlockSpec((1, din), lambda i, j: (0, 0)),
          pl.BlockSpec((1, din, dout), lambda i, j: (j, 0, 0)),
      ],
      out_specs=pl.BlockSpec((bn, dout), lambda i, j: (j * nb + i, 0)),
      out_shape=jax.ShapeDtypeStruct((nine * _N, dout), jnp.float32),
      scratch_shapes=[pltpu.VMEM((bn, din), jnp.float32)],
  )(a, a, mm_prev, b2, wstack)


def _tc_fuse(a, mm, b2, relu):
  width = a.shape[-1]
  bn = 2000
  nb = _N // bn

  def body(a0_ref, a1_ref, r_ref, b_ref, o_ref):
    z = a0_ref[0] + a1_ref[0] + r_ref[...] + b_ref[0]
    if relu:
      o_ref[...] = jnp.maximum(z, 0.0)
    else:
      m = jnp.max(z, axis=-1, keepdims=True)
      e = jnp.exp(z - m)
      o_ref[...] = z - m - jnp.log(jnp.sum(e, axis=-1, keepdims=True))

  return pl.pallas_call(
      body,
      grid=(nb,),
      in_specs=[
          pl.BlockSpec((1, bn, width), lambda i: (0, i, 0)),
          pl.BlockSpec((1, bn, width), lambda i: (1, i, 0)),
          pl.BlockSpec((bn, width), lambda i: (8 * nb + i, 0)),
          pl.BlockSpec((1, width), lambda i: (0, 0)),
      ],
      out_specs=pl.BlockSpec((bn, width), lambda i: (i, 0)),
      out_shape=jax.ShapeDtypeStruct((_N, width), jnp.float32),
  )(a, a, mm, b2)


def kernel(x, edge_index, edge_type,
           W1_rel, W1_root, b1,
           W2_rel, W2_root, b2,
           W3_rel, W3_root, b3,
           W4_rel, W4_root, b4):
  src = edge_index[0].astype(jnp.int32)
  dst = edge_index[1].astype(jnp.int32)
  et = edge_type.astype(jnp.int32)
  gidx2d = (et * _N + src).reshape(_E // _K, _K)
  seg2d = (dst * _R + et).reshape(_E // _K, _K)
  dst2d = dst.reshape(_E // _K, _K)

  cnt = _sc_counts(seg2d)
  wedge = _sc_wedge(cnt[0], cnt[1], seg2d)

  stacks = [jnp.concatenate([w_rel, w_root[None]], axis=0)
            for w_rel, w_root in ((W1_rel, W1_root), (W2_rel, W2_root),
                                  (W3_rel, W3_root), (W4_rel, W4_root))]
  mm = _tc_transform(x.astype(jnp.float32), stacks[0])
  a = _sc_edge_pass(mm, wedge, gidx2d, dst2d, 128)
  for li, b in ((1, b1), (2, b2), (3, b3)):
    mm = _tc_transform_fused(a, mm, b.reshape(1, -1), stacks[li])
    a = _sc_edge_pass(mm, wedge, gidx2d, dst2d, stacks[li].shape[-1])
  return _tc_fuse(a, mm, b4.reshape(1, -1), relu=False)
